# Initial kernel scaffold; baseline (speedup 1.0000x reference)
import jax
import jax.numpy as jnp
from jax.experimental import pallas as pl


def kernel(item_fea, W_brand, W_category):
    raise NotImplementedError("write your pallas kernel here")


# speedup vs baseline: 1.1178x; 1.1178x over previous
import functools

import jax
import jax.numpy as jnp
from jax import lax
from jax.experimental import pallas as pl
from jax.experimental.pallas import tpu as pltpu
from jax.experimental.pallas import tpu_sc as plsc

NC = 2
NS = 16
NW = NC * NS
LANES = 16
CH = 128


def _make_kernel(B, D):
    bpw = B // NW
    nch = bpw // CH
    mesh = plsc.VectorSubcoreMesh(core_axis_name="c", subcore_axis_name="s")

    @functools.partial(
        pl.kernel,
        mesh=mesh,
        compiler_params=pltpu.CompilerParams(
            needs_layout_passes=False, use_tc_tiling_on_sc=False),
        out_type=jax.ShapeDtypeStruct((2 * B, D), jnp.float32),
        scratch_types=[
            pltpu.VMEM((bpw * 3,), jnp.int32),
            pltpu.VMEM((nch, CH), jnp.int32),
            pltpu.VMEM((nch, CH), jnp.int32),
            pltpu.VMEM((nch, CH), jnp.int32),
            pltpu.VMEM((nch, CH), jnp.int32),
            pltpu.VMEM((nch, CH, D), jnp.float32),
            pltpu.VMEM((nch, CH, D), jnp.float32),
            pltpu.SemaphoreType.DMA,
            pltpu.SemaphoreType.DMA,
        ],
    )
    def body(fea_hbm, wb_hbm, wc_hbm, out_hbm,
             fea_v, bidx, cidx, eidx, oidx, brows, crows, semb, semc):
        wid = lax.axis_index("s") * NC + lax.axis_index("c")
        base = wid * bpw
        pltpu.sync_copy(fea_hbm.at[pl.ds(base * 3, bpw * 3)], fea_v)

        iota = lax.iota(jnp.int32, LANES)
        for c in range(bpw // LANES):
            flat = (iota + c * LANES) * 3
            b = plsc.load_gather(fea_v, [flat + 1])
            ct = plsc.load_gather(fea_v, [flat + 2])
            r = c // (CH // LANES)
            col = (c % (CH // LANES)) * LANES
            bidx[r, pl.ds(col, LANES)] = b
            cidx[r, pl.ds(col, LANES)] = ct
            gpos = (base + c * LANES) * 2 + iota * 2
            eidx[r, pl.ds(col, LANES)] = gpos
            oidx[r, pl.ds(col, LANES)] = gpos + 1

        for j in range(nch):
            pltpu.async_copy(wb_hbm.at[bidx.at[j]], brows.at[j], semb).wait()
            pltpu.async_copy(wc_hbm.at[cidx.at[j]], crows.at[j], semc).wait()
            pltpu.async_copy(brows.at[j], out_hbm.at[eidx.at[j]], semb).wait()
            pltpu.async_copy(crows.at[j], out_hbm.at[oidx.at[j]], semc).wait()

    return body


def kernel(item_fea, W_brand, W_category):
    B = item_fea.shape[0]
    D = W_brand.shape[1]
    out2 = _make_kernel(B, D)(item_fea.astype(jnp.int32).reshape(-1),
                              W_brand, W_category)
    return out2.reshape(B, 2 * D)


# speedup vs baseline: 1.1555x; 1.0337x over previous
import functools

import jax
import jax.numpy as jnp
from jax import lax
from jax.experimental import pallas as pl
from jax.experimental.pallas import tpu as pltpu
from jax.experimental.pallas import tpu_sc as plsc

NC = 2
NS = 16
NW = NC * NS
LANES = 16
CH = 128


def _make_kernel(B, D):
    bpw = B // NW
    nch = bpw // CH
    mesh = plsc.VectorSubcoreMesh(core_axis_name="c", subcore_axis_name="s")

    @functools.partial(
        pl.kernel,
        mesh=mesh,
        compiler_params=pltpu.CompilerParams(
            needs_layout_passes=False, use_tc_tiling_on_sc=False),
        out_type=jax.ShapeDtypeStruct((2 * B, D), jnp.float32),
        scratch_types=[
            pltpu.VMEM((bpw * 3,), jnp.int32),
            pltpu.VMEM((nch, CH), jnp.int32),
            pltpu.VMEM((nch, CH), jnp.int32),
            pltpu.VMEM((nch, CH), jnp.int32),
            pltpu.VMEM((nch, CH), jnp.int32),
            pltpu.VMEM((nch, CH, D), jnp.float32),
            pltpu.VMEM((nch, CH, D), jnp.float32),
        ] + [pltpu.SemaphoreType.DMA] * (2 * (B // NW // CH)),
    )
    def body(fea_hbm, wb_hbm, wc_hbm, out_hbm,
             fea_v, bidx, cidx, eidx, oidx, brows, crows, *sems):
        wid = lax.axis_index("s") * NC + lax.axis_index("c")
        base = wid * bpw
        pltpu.sync_copy(fea_hbm.at[pl.ds(base * 3, bpw * 3)], fea_v)

        iota = lax.iota(jnp.int32, LANES)
        gathers = []
        for j in range(nch):
            for cc in range(CH // LANES):
                c = j * (CH // LANES) + cc
                flat = (iota + c * LANES) * 3
                b = plsc.load_gather(fea_v, [flat + 1])
                ct = plsc.load_gather(fea_v, [flat + 2])
                col = cc * LANES
                bidx[j, pl.ds(col, LANES)] = b
                cidx[j, pl.ds(col, LANES)] = ct
                gpos = (base + c * LANES) * 2 + iota * 2
                eidx[j, pl.ds(col, LANES)] = gpos
                oidx[j, pl.ds(col, LANES)] = gpos + 1
            gathers.append((
                pltpu.async_copy(wb_hbm.at[bidx.at[j]], brows.at[j],
                                 sems[2 * j]),
                pltpu.async_copy(wc_hbm.at[cidx.at[j]], crows.at[j],
                                 sems[2 * j + 1]),
            ))
        scatters = []
        for j in range(nch):
            gb, gc = gathers[j]
            gb.wait()
            scatters.append(pltpu.async_copy(
                brows.at[j], out_hbm.at[eidx.at[j]], sems[2 * j]))
            gc.wait()
            scatters.append(pltpu.async_copy(
                crows.at[j], out_hbm.at[oidx.at[j]], sems[2 * j + 1]))
        for s in scatters:
            s.wait()

    return body


def kernel(item_fea, W_brand, W_category):
    B = item_fea.shape[0]
    D = W_brand.shape[1]
    out2 = _make_kernel(B, D)(item_fea.astype(jnp.int32).reshape(-1),
                              W_brand, W_category)
    return out2.reshape(B, 2 * D)
